# Initial kernel scaffold; baseline (speedup 1.0000x reference)
import jax
import jax.numpy as jnp
from jax.experimental import pallas as pl


def kernel(coords, atom_idx, element_idx, residue_type, residue_ids, atom_emb, elem_emb, res_emb, W_in, Wq, Wk, Wv, Wo, B_bias, W1, W2, W_outrep, W_proj, b_proj):
    raise NotImplementedError("write your pallas kernel here")


# speedup vs baseline: 58.9676x; 58.9676x over previous
import functools

import jax
import jax.numpy as jnp
from jax import lax
from jax.experimental import pallas as pl
from jax.experimental.pallas import tpu as pltpu
from jax.experimental.pallas import tpu_sc as plsc

N = 4096
K = 16
E = N * K
NUM_RES = 200
D_H = 64
H = 4
DH = D_H // H
BINS = 100
L = 4
D_OUT = 16
OUT_CH = 2
MIN_D = 0.0
MAX_D = 10.0

_INV_SQRT_DH = 1.0 / (DH ** 0.5)

_SC_CORES = 2
_SC_SUBCORES = 16
_NW = _SC_CORES * _SC_SUBCORES
_CHUNK = 128


def _embed_body(aidx_ref, ridx_ref, eidx_ref, a_emb_ref, r_emb_ref, e_emb_ref,
                w_in_ref, out_ref):
    aidx = aidx_ref[:]
    ridx = ridx_ref[:]
    eidx = eidx_ref[:]
    oh_a = (lax.broadcasted_iota(jnp.int32, (N, 85), 1) == aidx).astype(jnp.float32)
    oh_r = (lax.broadcasted_iota(jnp.int32, (N, 4), 1) == ridx).astype(jnp.float32)
    oh_e = (lax.broadcasted_iota(jnp.int32, (N, 10), 1) == eidx).astype(jnp.float32)
    ea = jnp.dot(oh_a, a_emb_ref[:], preferred_element_type=jnp.float32)
    er = jnp.dot(oh_r, r_emb_ref[:], preferred_element_type=jnp.float32)
    ee = jnp.dot(oh_e, e_emb_ref[:], preferred_element_type=jnp.float32)
    w = w_in_ref[:]
    out_ref[:] = (jnp.dot(ea, w[0:16, :], preferred_element_type=jnp.float32)
                  + jnp.dot(er, w[16:24, :], preferred_element_type=jnp.float32)
                  + jnp.dot(ee, w[24:32, :], preferred_element_type=jnp.float32))


def _embed_call(atom_idx, residue_type, element_idx, atom_emb, res_emb,
                elem_emb, w_in):
    return pl.pallas_call(
        _embed_body,
        out_shape=jax.ShapeDtypeStruct((N, D_H), jnp.float32),
    )(atom_idx.reshape(N, 1).astype(jnp.int32),
      residue_type.reshape(N, 1).astype(jnp.int32),
      element_idx.reshape(N, 1).astype(jnp.int32),
      atom_emb.astype(jnp.float32), res_emb.astype(jnp.float32),
      elem_emb.astype(jnp.float32), w_in.astype(jnp.float32))


_KNN_BLK = 256


def _knn_body(cb_ref, call_ref, idx_ref):
    i = pl.program_id(0)
    cb = cb_ref[:]
    ca = call_ref[:]
    x2b = jnp.sum(cb * cb, axis=1, keepdims=True)
    x2a = jnp.sum(ca * ca, axis=1)[None, :]
    ab = lax.dot_general(cb, ca, (((1,), (1,)), ((), ())),
                         preferred_element_type=jnp.float32)
    d2 = x2b + x2a - 2.0 * ab
    colio = lax.broadcasted_iota(jnp.int32, (_KNN_BLK, N), 1)
    rowio = lax.broadcasted_iota(jnp.int32, (_KNN_BLK, N), 0) + i * _KNN_BLK
    d2 = jnp.where(colio == rowio, jnp.inf, d2)
    idxs = []
    for _ in range(K):
        m = jnp.min(d2, axis=1, keepdims=True)
        cand = jnp.where(d2 == m, colio, N)
        am = jnp.min(cand, axis=1, keepdims=True)
        idxs.append(am)
        d2 = jnp.where(colio == am, jnp.inf, d2)
    idx_ref[:] = jnp.concatenate(idxs, axis=1)


def _knn_call(coords4):
    return pl.pallas_call(
        _knn_body,
        grid=(N // _KNN_BLK,),
        in_specs=[pl.BlockSpec((_KNN_BLK, 4), lambda i: (i, 0)),
                  pl.BlockSpec((N, 4), lambda i: (0, 0))],
        out_specs=pl.BlockSpec((_KNN_BLK, K), lambda i: (i, 0)),
        out_shape=jax.ShapeDtypeStruct((N, K), jnp.int32),
    )(coords4, coords4)


def _sc_gather(table, idx_flat, d):
    rows_per_w = E // _NW
    nchunk = rows_per_w // _CHUNK
    idx3 = idx_flat.reshape(_NW, nchunk, _CHUNK).astype(jnp.int32)
    mesh = plsc.VectorSubcoreMesh(core_axis_name="c", subcore_axis_name="s",
                                  num_cores=_SC_CORES,
                                  num_subcores=_SC_SUBCORES)

    @functools.partial(
        pl.kernel,
        out_type=jax.ShapeDtypeStruct((E, d), jnp.float32),
        mesh=mesh,
        scratch_types=[
            pltpu.VMEM((nchunk, _CHUNK), jnp.int32),
            pltpu.VMEM((_CHUNK, d), jnp.float32),
            pltpu.VMEM((_CHUNK, d), jnp.float32),
            pltpu.SemaphoreType.DMA,
            pltpu.SemaphoreType.DMA,
        ],
        compiler_params=pltpu.CompilerParams(use_tc_tiling_on_sc=False),
    )
    def k(table_hbm, idx_hbm, out_hbm, idx_v, buf0, buf1, sem0, sem1):
        wid = lax.axis_index("s") * _SC_CORES + lax.axis_index("c")
        base = wid * rows_per_w
        pltpu.sync_copy(idx_hbm.at[wid], idx_v)
        bufs = (buf0, buf1)
        sems = (sem0, sem1)
        copies = []
        for c in range(nchunk):
            cp = pltpu.async_copy(table_hbm.at[idx_v.at[c]], bufs[c % 2],
                                  sems[c % 2])
            if c >= 1:
                copies[c - 1].wait()
                pltpu.sync_copy(bufs[(c - 1) % 2],
                                out_hbm.at[pl.ds(base + (c - 1) * _CHUNK,
                                                 _CHUNK)])
            copies.append(cp)
        copies[-1].wait()
        pltpu.sync_copy(bufs[(nchunk - 1) % 2],
                        out_hbm.at[pl.ds(base + (nchunk - 1) * _CHUNK,
                                         _CHUNK)])

    return k(table, idx3)


_BIAS_BLK = 256


def _bias_body(cb_ref, gc_ref, bcat_ref, out_ref):
    eb = _BIAS_BLK * K
    cb = cb_ref[:]
    gc = gc_ref[:]
    cd = jnp.broadcast_to(cb[:, None, :], (_BIAS_BLK, K, 4)).reshape(eb, 4)
    diff = cd - gc
    d2 = jnp.sum(diff * diff, axis=1, keepdims=True)
    dist = jnp.sqrt(jnp.maximum(d2, 1e-12))
    step = (MAX_D - MIN_D) / (BINS - 1)
    centers = (MIN_D + step * lax.broadcasted_iota(
        jnp.int32, (eb, BINS), 1).astype(jnp.float32))
    sigma = (MAX_D - MIN_D) / BINS
    z = dist - centers
    rbf = jnp.exp(z * z * (-1.0 / (2.0 * sigma * sigma)))
    out_ref[:] = jnp.dot(rbf, bcat_ref[:], preferred_element_type=jnp.float32)


def _bias_call(coords4, gcoords, bcat):
    eb = _BIAS_BLK * K
    return pl.pallas_call(
        _bias_body,
        grid=(N // _BIAS_BLK,),
        in_specs=[pl.BlockSpec((_BIAS_BLK, 4), lambda i: (i, 0)),
                  pl.BlockSpec((eb, 4), lambda i: (i, 0)),
                  pl.BlockSpec((BINS, L * H), lambda i: (0, 0))],
        out_specs=pl.BlockSpec((eb, L * H), lambda i: (i, 0)),
        out_shape=jax.ShapeDtypeStruct((E, L * H), jnp.float32),
    )(coords4, gcoords, bcat)


_ATT_BLK = 512


def _layer_body(h_ref, gh_ref, bias_ref, wq_ref, wk_ref, wv_ref, wo_ref,
                w1_ref, w2_ref, out_ref):
    eb = _ATT_BLK * K
    hb = h_ref[:]
    q = jnp.dot(hb, wq_ref[:], preferred_element_type=jnp.float32)
    hg = gh_ref[:]
    ke = jnp.dot(hg, wk_ref[:], preferred_element_type=jnp.float32)
    ve = jnp.dot(hg, wv_ref[:], preferred_element_type=jnp.float32)
    qexp = jnp.broadcast_to(q[:, None, :], (_ATT_BLK, K, D_H)).reshape(eb, D_H)
    g = (lax.broadcasted_iota(jnp.int32, (D_H, H), 0) // DH
         == lax.broadcasted_iota(jnp.int32, (D_H, H), 1)).astype(jnp.float32)
    logits = (jnp.dot(qexp * ke, g, preferred_element_type=jnp.float32)
              * _INV_SQRT_DH + bias_ref[:])
    lg = logits.reshape(_ATT_BLK, K, H)
    m = jnp.max(lg, axis=1, keepdims=True)
    ex = jnp.exp(lg - m)
    den = jnp.sum(ex, axis=1, keepdims=True)
    alpha = (ex / (den + 1e-9)).reshape(eb, H)
    gt = (lax.broadcasted_iota(jnp.int32, (H, D_H), 0)
          == lax.broadcasted_iota(jnp.int32, (H, D_H), 1) // DH
          ).astype(jnp.float32)
    aexp = jnp.dot(alpha, gt, preferred_element_type=jnp.float32)
    agg = (aexp * ve).reshape(_ATT_BLK, K, D_H).sum(axis=1)
    hb2 = hb + jnp.dot(agg, wo_ref[:], preferred_element_type=jnp.float32)
    ffn = jnp.dot(
        jax.nn.relu(jnp.dot(hb2, w1_ref[:], preferred_element_type=jnp.float32)),
        w2_ref[:], preferred_element_type=jnp.float32)
    out_ref[:] = hb2 + ffn


def _layer_call(h, gh, bias_l, wq, wk, wv, wo, w1, w2):
    eb = _ATT_BLK * K
    return pl.pallas_call(
        _layer_body,
        grid=(N // _ATT_BLK,),
        in_specs=[pl.BlockSpec((_ATT_BLK, D_H), lambda i: (i, 0)),
                  pl.BlockSpec((eb, D_H), lambda i: (i, 0)),
                  pl.BlockSpec((eb, H), lambda i: (i, 0)),
                  pl.BlockSpec((D_H, D_H), lambda i: (0, 0)),
                  pl.BlockSpec((D_H, D_H), lambda i: (0, 0)),
                  pl.BlockSpec((D_H, D_H), lambda i: (0, 0)),
                  pl.BlockSpec((D_H, D_H), lambda i: (0, 0)),
                  pl.BlockSpec((D_H, 2 * D_H), lambda i: (0, 0)),
                  pl.BlockSpec((2 * D_H, D_H), lambda i: (0, 0))],
        out_specs=pl.BlockSpec((_ATT_BLK, D_H), lambda i: (i, 0)),
        out_shape=jax.ShapeDtypeStruct((N, D_H), jnp.float32),
    )(h, gh, bias_l, wq, wk, wv, wo, w1, w2)


def _readout_body(h_ref, rid_ref, wrep_ref, wproj_ref, bproj_ref, out_ref):
    oh = (lax.broadcasted_iota(jnp.int32, (NUM_RES, N), 0)
          == rid_ref[:]).astype(jnp.float32)
    atoms = jnp.dot(h_ref[:], wrep_ref[:], preferred_element_type=jnp.float32)
    seg = jnp.dot(oh, atoms, preferred_element_type=jnp.float32)
    counts = jnp.sum(oh, axis=1, keepdims=True)
    res_feat = seg / jnp.maximum(counts, 1.0)
    out_ref[:] = (jnp.dot(res_feat, wproj_ref[:],
                          preferred_element_type=jnp.float32) + bproj_ref[:])


def _readout_call(h, residue_ids, w_outrep, w_proj, b_proj):
    return pl.pallas_call(
        _readout_body,
        out_shape=jax.ShapeDtypeStruct((NUM_RES, OUT_CH), jnp.float32),
    )(h, residue_ids.reshape(1, N).astype(jnp.int32),
      w_outrep.astype(jnp.float32), w_proj.astype(jnp.float32),
      b_proj.reshape(1, OUT_CH).astype(jnp.float32))


def kernel(coords, atom_idx, element_idx, residue_type, residue_ids, atom_emb,
           elem_emb, res_emb, W_in, Wq, Wk, Wv, Wo, B_bias, W1, W2, W_outrep,
           W_proj, b_proj):
    coords = coords.astype(jnp.float32)
    coords4 = jnp.pad(coords, ((0, 0), (0, 1)))
    h = _embed_call(atom_idx, residue_type, element_idx, atom_emb, res_emb,
                    elem_emb, W_in)
    idx = _knn_call(coords4)
    src = idx.reshape(E)
    gcoords = _sc_gather(coords4, src, 4)
    bcat = jnp.transpose(B_bias.astype(jnp.float32), (1, 0, 2)).reshape(
        BINS, L * H)
    bias_all = _bias_call(coords4, gcoords, bcat)
    for l in range(L):
        gh = _sc_gather(h, src, D_H)
        bias_l = lax.slice(bias_all, (0, l * H), (E, (l + 1) * H))
        h = _layer_call(h, gh, bias_l,
                        Wq[l].astype(jnp.float32), Wk[l].astype(jnp.float32),
                        Wv[l].astype(jnp.float32), Wo[l].astype(jnp.float32),
                        W1[l].astype(jnp.float32), W2[l].astype(jnp.float32))
    return _readout_call(h, residue_ids, W_outrep, W_proj, b_proj)
